# baseline (speedup 1.0000x reference)
import functools

import jax
import jax.numpy as jnp
from jax.experimental import pallas as pl


def _body(pref, suf, opref, osuf, ctx_r, ctxo_r, out, *, rows, cls_blocks, n_ctx, ctx_dim):
    i = pl.program_id(0)
    c = ctx_dim
    k = n_ctx * ctx_dim

    @pl.when(i < cls_blocks)
    def _():
        out[0, :, 0:c] = pref[0]
        out[0, :, c:c + k] = jnp.broadcast_to(ctx_r[0], (rows, k))
        out[0, :, c + k:] = suf[0]

    @pl.when(i >= cls_blocks)
    def _():
        out[0, :, 0:c] = opref[0]
        out[0, :, c:c + k] = jnp.broadcast_to(ctxo_r[0], (rows, k))
        out[0, :, c + k:] = osuf[0]


def kernel(ctx, ctx_ood, token_prefix, token_suffix, ood_token_prefix, ood_token_suffix):
    n_cls = token_prefix.shape[0]
    n_ex = ood_token_prefix.shape[0]
    n_ctx, ctx_dim = ctx.shape
    suf_len = token_suffix.shape[1]
    seq = 1 + n_ctx + suf_len
    row_w = seq * ctx_dim
    suf_w = suf_len * ctx_dim
    ctx_w = n_ctx * ctx_dim

    rows = 20
    cls_blocks = n_cls // rows
    ex_blocks = n_ex // rows
    grid = (cls_blocks + ex_blocks,)

    pref3 = token_prefix.reshape(cls_blocks, rows, ctx_dim)
    suf3 = token_suffix.reshape(cls_blocks, rows, suf_w)
    opref3 = ood_token_prefix.reshape(ex_blocks, rows, ctx_dim)
    osuf3 = ood_token_suffix.reshape(ex_blocks, rows, suf_w)
    ctx3 = ctx.reshape(1, 1, ctx_w)
    ctxo3 = ctx_ood.reshape(1, 1, ctx_w)

    cls_idx = lambda i: (jnp.minimum(i, cls_blocks - 1), 0, 0)
    ood_idx = lambda i: (jnp.maximum(i - cls_blocks, 0), 0, 0)

    out = pl.pallas_call(
        functools.partial(_body, rows=rows, cls_blocks=cls_blocks,
                          n_ctx=n_ctx, ctx_dim=ctx_dim),
        grid=grid,
        in_specs=[
            pl.BlockSpec((1, rows, ctx_dim), cls_idx),
            pl.BlockSpec((1, rows, suf_w), cls_idx),
            pl.BlockSpec((1, rows, ctx_dim), ood_idx),
            pl.BlockSpec((1, rows, suf_w), ood_idx),
            pl.BlockSpec((1, 1, ctx_w), lambda i: (0, 0, 0)),
            pl.BlockSpec((1, 1, ctx_w), lambda i: (0, 0, 0)),
        ],
        out_specs=pl.BlockSpec((1, rows, row_w), lambda i: (i, 0, 0)),
        out_shape=jax.ShapeDtypeStruct((cls_blocks + ex_blocks, rows, row_w), ctx.dtype),
    )(pref3, suf3, opref3, osuf3, ctx3, ctxo3)
    return out.reshape(n_cls + n_ex, seq, ctx_dim)


# speedup vs baseline: 2.4461x; 2.4461x over previous
import functools

import jax
import jax.numpy as jnp
from jax.experimental import pallas as pl


def _body(pref, suf, opref, osuf, ctx_r, ctxo_r, out, *, rows, cls_blocks, n_ctx):
    i = pl.program_id(0)

    @pl.when(i < cls_blocks)
    def _():
        out[:, 0:1, :] = pref[...]
        out[:, 1:1 + n_ctx, :] = jnp.broadcast_to(ctx_r[...][None], (rows,) + ctx_r.shape)
        out[:, 1 + n_ctx:, :] = suf[...]

    @pl.when(i >= cls_blocks)
    def _():
        out[:, 0:1, :] = opref[...]
        out[:, 1:1 + n_ctx, :] = jnp.broadcast_to(ctxo_r[...][None], (rows,) + ctxo_r.shape)
        out[:, 1 + n_ctx:, :] = osuf[...]


def kernel(ctx, ctx_ood, token_prefix, token_suffix, ood_token_prefix, ood_token_suffix):
    n_cls = token_prefix.shape[0]
    n_ex = ood_token_prefix.shape[0]
    n_ctx, ctx_dim = ctx.shape
    suf_len = token_suffix.shape[1]
    seq = 1 + n_ctx + suf_len

    rows = 20
    cls_blocks = n_cls // rows
    ex_blocks = n_ex // rows

    cls_idx = lambda i: (jnp.minimum(i, cls_blocks - 1), 0, 0)
    ood_idx = lambda i: (jnp.maximum(i - cls_blocks, 0), 0, 0)

    return pl.pallas_call(
        functools.partial(_body, rows=rows, cls_blocks=cls_blocks, n_ctx=n_ctx),
        grid=(cls_blocks + ex_blocks,),
        in_specs=[
            pl.BlockSpec((rows, 1, ctx_dim), cls_idx),
            pl.BlockSpec((rows, suf_len, ctx_dim), cls_idx),
            pl.BlockSpec((rows, 1, ctx_dim), ood_idx),
            pl.BlockSpec((rows, suf_len, ctx_dim), ood_idx),
            pl.BlockSpec((n_ctx, ctx_dim), lambda i: (0, 0)),
            pl.BlockSpec((n_ctx, ctx_dim), lambda i: (0, 0)),
        ],
        out_specs=pl.BlockSpec((rows, seq, ctx_dim), lambda i: (i, 0, 0)),
        out_shape=jax.ShapeDtypeStruct((n_cls + n_ex, seq, ctx_dim), ctx.dtype),
    )(token_prefix, token_suffix, ood_token_prefix, ood_token_suffix, ctx, ctx_ood)


# speedup vs baseline: 7.7908x; 3.1850x over previous
import functools

import jax
import jax.numpy as jnp
from jax.experimental import pallas as pl


def _body(pref, opref, suf, osuf, ctx_b, ctxo_b, out, *, n_cls, n_ex, n_ctx):
    i = pl.program_id(0)

    @pl.when(i == 0)
    def _():
        out[0, 0:n_cls, :] = pref[0]
        out[0, n_cls:, :] = opref[0]

    @pl.when(jnp.logical_and(i >= 1, i < 1 + n_ctx))
    def _():
        out[0, 0:n_cls, :] = jnp.broadcast_to(ctx_b[0], (n_cls, ctx_b.shape[2]))
        out[0, n_cls:, :] = jnp.broadcast_to(ctxo_b[0], (n_ex, ctxo_b.shape[2]))

    @pl.when(i >= 1 + n_ctx)
    def _():
        out[0, 0:n_cls, :] = suf[0]
        out[0, n_cls:, :] = osuf[0]


def kernel(ctx, ctx_ood, token_prefix, token_suffix, ood_token_prefix, ood_token_suffix):
    n_cls = token_prefix.shape[0]
    n_ex = ood_token_prefix.shape[0]
    n_ctx, ctx_dim = ctx.shape
    suf_len = token_suffix.shape[1]
    seq = 1 + n_ctx + suf_len

    prefT = token_prefix.transpose(1, 0, 2)
    oprefT = ood_token_prefix.transpose(1, 0, 2)
    sufT = token_suffix.transpose(1, 0, 2)
    osufT = ood_token_suffix.transpose(1, 0, 2)
    ctx3 = ctx.reshape(n_ctx, 1, ctx_dim)
    ctxo3 = ctx_ood.reshape(n_ctx, 1, ctx_dim)

    suf_idx = lambda i: (jnp.clip(i - (1 + n_ctx), 0, suf_len - 1), 0, 0)
    ctx_idx = lambda i: (jnp.clip(i - 1, 0, n_ctx - 1), 0, 0)
    zero_idx = lambda i: (0, 0, 0)

    outT = pl.pallas_call(
        functools.partial(_body, n_cls=n_cls, n_ex=n_ex, n_ctx=n_ctx),
        grid=(seq,),
        in_specs=[
            pl.BlockSpec((1, n_cls, ctx_dim), zero_idx),
            pl.BlockSpec((1, n_ex, ctx_dim), zero_idx),
            pl.BlockSpec((1, n_cls, ctx_dim), suf_idx),
            pl.BlockSpec((1, n_ex, ctx_dim), suf_idx),
            pl.BlockSpec((1, 1, ctx_dim), ctx_idx),
            pl.BlockSpec((1, 1, ctx_dim), ctx_idx),
        ],
        out_specs=pl.BlockSpec((1, n_cls + n_ex, ctx_dim), lambda i: (i, 0, 0)),
        out_shape=jax.ShapeDtypeStruct((seq, n_cls + n_ex, ctx_dim), ctx.dtype),
    )(prefT, oprefT, sufT, osufT, ctx3, ctxo3)
    return outT.transpose(1, 0, 2)


# speedup vs baseline: 8.3806x; 1.0757x over previous
import functools

import jax
import jax.numpy as jnp
from jax.experimental import pallas as pl


def _body(pref, opref, suf, osuf, ctx_v, ctxo_v, out, *, n_cls, n_ex, n_ctx):
    i = pl.program_id(0)

    @pl.when(i == 0)
    def _():
        out[0, 0:n_cls, :] = pref[:, 0, :]
        out[0, n_cls:, :] = opref[:, 0, :]

    @pl.when(jnp.logical_and(i >= 1, i < 1 + n_ctx))
    def _():
        j = jnp.clip(i - 1, 0, n_ctx - 1)
        out[0, 0:n_cls, :] = jnp.broadcast_to(ctx_v[pl.ds(j, 1), :], (n_cls, ctx_v.shape[1]))
        out[0, n_cls:, :] = jnp.broadcast_to(ctxo_v[pl.ds(j, 1), :], (n_ex, ctxo_v.shape[1]))

    @pl.when(i >= 1 + n_ctx)
    def _():
        out[0, 0:n_cls, :] = suf[0]
        out[0, n_cls:, :] = osuf[0]


def kernel(ctx, ctx_ood, token_prefix, token_suffix, ood_token_prefix, ood_token_suffix):
    n_cls = token_prefix.shape[0]
    n_ex = ood_token_prefix.shape[0]
    n_ctx, ctx_dim = ctx.shape
    suf_len = token_suffix.shape[1]
    seq = 1 + n_ctx + suf_len

    sufT = token_suffix.transpose(1, 0, 2)
    osufT = ood_token_suffix.transpose(1, 0, 2)

    suf_idx = lambda i: (jnp.clip(i - (1 + n_ctx), 0, suf_len - 1), 0, 0)
    zero_idx = lambda i: (0, 0, 0)

    outT = pl.pallas_call(
        functools.partial(_body, n_cls=n_cls, n_ex=n_ex, n_ctx=n_ctx),
        grid=(seq,),
        in_specs=[
            pl.BlockSpec((n_cls, 1, ctx_dim), zero_idx),
            pl.BlockSpec((n_ex, 1, ctx_dim), zero_idx),
            pl.BlockSpec((1, n_cls, ctx_dim), suf_idx),
            pl.BlockSpec((1, n_ex, ctx_dim), suf_idx),
            pl.BlockSpec((n_ctx, ctx_dim), lambda i: (0, 0)),
            pl.BlockSpec((n_ctx, ctx_dim), lambda i: (0, 0)),
        ],
        out_specs=pl.BlockSpec((1, n_cls + n_ex, ctx_dim), lambda i: (i, 0, 0)),
        out_shape=jax.ShapeDtypeStruct((seq, n_cls + n_ex, ctx_dim), ctx.dtype),
    )(token_prefix, ood_token_prefix, sufT, osufT, ctx, ctx_ood)
    return outT.transpose(1, 0, 2)


# speedup vs baseline: 8.9295x; 1.0655x over previous
import functools

import jax
import jax.numpy as jnp
from jax.experimental import pallas as pl


def _sub_slab(r, slab, pref, opref, sufr, osufr, ctx_v, ctxo_v, out, n_cls, n_ctx):
    @pl.when(slab == 0)
    def _():
        out[r, 0:n_cls, :] = pref[:, 0, :]
        out[r, n_cls:, :] = opref[:, 0, :]

    @pl.when(jnp.logical_and(slab >= 1, slab < 1 + n_ctx))
    def _():
        j = jnp.clip(slab - 1, 0, n_ctx - 1)
        out[r, 0:n_cls, :] = jnp.broadcast_to(ctx_v[pl.ds(j, 1), :], (n_cls, ctx_v.shape[1]))
        out[r, n_cls:, :] = jnp.broadcast_to(ctxo_v[pl.ds(j, 1), :], (out.shape[1] - n_cls, ctxo_v.shape[1]))

    @pl.when(slab >= 1 + n_ctx)
    def _():
        out[r, 0:n_cls, :] = sufr[0]
        out[r, n_cls:, :] = osufr[0]


def _body(pref, opref, sufa, sufb, osufa, osufb, ctx_v, ctxo_v, out, *, n_cls, n_ctx):
    b = pl.program_id(0)
    _sub_slab(0, 2 * b, pref, opref, sufa, osufa, ctx_v, ctxo_v, out, n_cls, n_ctx)
    _sub_slab(1, 2 * b + 1, pref, opref, sufb, osufb, ctx_v, ctxo_v, out, n_cls, n_ctx)


def kernel(ctx, ctx_ood, token_prefix, token_suffix, ood_token_prefix, ood_token_suffix):
    n_cls = token_prefix.shape[0]
    n_ex = ood_token_prefix.shape[0]
    n_ctx, ctx_dim = ctx.shape
    suf_len = token_suffix.shape[1]
    seq = 1 + n_ctx + suf_len
    s0 = 1 + n_ctx

    sufT = token_suffix.transpose(1, 0, 2)
    osufT = ood_token_suffix.transpose(1, 0, 2)

    sufa_idx = lambda b: (jnp.clip(2 * b - s0, 0, suf_len - 1), 0, 0)
    sufb_idx = lambda b: (jnp.clip(2 * b + 1 - s0, 0, suf_len - 1), 0, 0)
    zero_idx = lambda b: (0, 0, 0)

    outT = pl.pallas_call(
        functools.partial(_body, n_cls=n_cls, n_ctx=n_ctx),
        grid=((seq + 1) // 2,),
        in_specs=[
            pl.BlockSpec((n_cls, 1, ctx_dim), zero_idx),
            pl.BlockSpec((n_ex, 1, ctx_dim), zero_idx),
            pl.BlockSpec((1, n_cls, ctx_dim), sufa_idx),
            pl.BlockSpec((1, n_cls, ctx_dim), sufb_idx),
            pl.BlockSpec((1, n_ex, ctx_dim), sufa_idx),
            pl.BlockSpec((1, n_ex, ctx_dim), sufb_idx),
            pl.BlockSpec((n_ctx, ctx_dim), lambda b: (0, 0)),
            pl.BlockSpec((n_ctx, ctx_dim), lambda b: (0, 0)),
        ],
        out_specs=pl.BlockSpec((2, n_cls + n_ex, ctx_dim), lambda b: (b, 0, 0)),
        out_shape=jax.ShapeDtypeStruct((seq, n_cls + n_ex, ctx_dim), ctx.dtype),
    )(token_prefix, ood_token_prefix, sufT, sufT, osufT, osufT, ctx, ctx_ood)
    return outT.transpose(1, 0, 2)
